# baseline (speedup 1.0000x reference)
import functools

import jax
import jax.numpy as jnp
from jax import lax
from jax.experimental import pallas as pl
from jax.experimental.pallas import tpu as pltpu
from jax.experimental.pallas import tpu_sc as plsc

H = 128
N = 10000
E = 320000
SPLITS = (32000, 64000, 96000, 96000, 32000)

NC = 2
NS = 16
NW = NC * NS
NP = 10240
ROWS_PER_TILE = NP // NS

_f32 = jnp.float32
_bf16 = jnp.bfloat16


def _pick_chunk(n):
    for c in range(128, 0, -8):
        if n % c == 0:
            return c
    raise ValueError(n)


def _precompute_body(x, w1s, w1r, b1, ps, pr):
    xv = x[...]
    ps[...] = jnp.dot(xv, w1s[...], preferred_element_type=_f32) + b1[...]
    pr[...] = jnp.dot(xv, w1r[...], preferred_element_type=_f32)


def _bdot(a, b):
    return jnp.dot(a.astype(_bf16), b, preferred_element_type=_f32)


def _edge_mlp_body(gs, gr, ea, w1e, w2, b2, w3, b3, w4, b4, g, beta,
                   ue, oe):
    eav = ea[...]
    h = (gs[...] + gr[...] + _bdot(eav, w1e[...]))
    h = jnp.maximum(h, 0.0)
    h = jnp.maximum(_bdot(h, w2[...]) + b2[...], 0.0)
    h = jnp.maximum(_bdot(h, w3[...]) + b3[...], 0.0)
    h = _bdot(h, w4[...]) + b4[...]
    mu = jnp.mean(h, axis=1, keepdims=True)
    d = h - mu
    var = jnp.mean(d * d, axis=1, keepdims=True)
    u = d * lax.rsqrt(var + 1e-5) * g[...] + beta[...]
    ue[...] = u
    oe[...] = eav + u


def _edge_mlp_body2(gs, gr, ea, w1e, w2, b2, w3, b3, w4, b4, g, beta, _oe_in,
                    ue, oe):
    _edge_mlp_body(gs, gr, ea, w1e, w2, b2, w3, b3, w4, b4, g, beta, ue, oe)


_NPART = 2 * len(SPLITS)


def _node_mlp_body(x, *args):
    parts = args[:_NPART]
    v1a, v1b, c1, v2, c2, v3, c3, v4, c4, gn, bn, out = args[_NPART:]
    xv = x[...]
    agg = parts[0][...]
    for p in parts[1:]:
        agg = agg + p[...]
    h = (jnp.dot(xv, v1a[...], preferred_element_type=_f32)
         + jnp.dot(agg, v1b[...], preferred_element_type=_f32) + c1[...])
    h = jnp.maximum(h, 0.0)
    h = jnp.maximum(jnp.dot(h, v2[...], preferred_element_type=_f32) + c2[...], 0.0)
    h = jnp.maximum(jnp.dot(h, v3[...], preferred_element_type=_f32) + c3[...], 0.0)
    h = jnp.dot(h, v4[...], preferred_element_type=_f32) + c4[...]
    mu = jnp.mean(h, axis=1, keepdims=True)
    d = h - mu
    var = jnp.mean(d * d, axis=1, keepdims=True)
    out[...] = xv + d * lax.rsqrt(var + 1e-5) * gn[...] + bn[...]


def _row_spec(block_rows, off=0):
    return pl.BlockSpec((block_rows, H), lambda i: (i + off, 0))


def _const_spec(shape):
    return pl.BlockSpec(shape, lambda i: (0, 0))


@functools.cache
def _sc_kernels(ne):
    epw = ne // NW
    chunk = _pick_chunk(epw)
    nchunk = epw // chunk
    mesh = plsc.VectorSubcoreMesh(core_axis_name="c", subcore_axis_name="s",
                                  num_cores=NC, num_subcores=NS)

    ept = ne // NS
    cht = _pick_chunk(ept)
    ncht = ept // cht

    @functools.partial(
        pl.kernel,
        out_type=[jax.ShapeDtypeStruct((ne, H), _f32),
                  jax.ShapeDtypeStruct((ne, H), _f32)],
        mesh=mesh,
        scratch_types=[
            pltpu.VMEM((ept,), jnp.int32),
            pltpu.VMEM((cht, H), _f32),
            pltpu.VMEM((cht, H), _f32),
            pltpu.VMEM_SHARED((NP, H), _f32),
        ] + [pltpu.SemaphoreType.DMA] * 4,
    )
    def sc_gather(ps_hbm, pr_hbm, send_hbm, recv_hbm, gs_hbm, gr_hbm,
                  idx, r0, r1, tbl, gsem0, gsem1, wsem0, wsem1):
        cid = lax.axis_index("c")
        sid = lax.axis_index("s")
        base = sid * ept

        def run_stream(tbl_hbm, sidx_hbm, out_hbm):
            pltpu.sync_copy(tbl_hbm.at[pl.ds(sid * ROWS_PER_TILE,
                                             ROWS_PER_TILE)],
                            tbl.at[pl.ds(sid * ROWS_PER_TILE,
                                         ROWS_PER_TILE)])
            pltpu.sync_copy(sidx_hbm.at[pl.ds(base, ept)], idx)
            plsc.subcore_barrier()

            def fire(c, r, sem):
                off = pl.multiple_of(c * cht, 8)
                pltpu.async_copy(tbl.at[idx.at[pl.ds(off, cht)]], r, sem)

            def wait_fire(r, sem):
                pltpu.make_async_copy(tbl.at[pl.ds(0, cht)], r, sem).wait()

            def wb(c, r, sem):
                off = pl.multiple_of(base + c * cht, 8)
                pltpu.async_copy(r, out_hbm.at[pl.ds(off, cht)], sem)

            def wait_wb(r, sem):
                pltpu.make_async_copy(r, out_hbm.at[pl.ds(0, cht)],
                                      sem).wait()

            fire(0, r0, gsem0)
            fire(1, r1, gsem1)

            def body(i, carry):
                c0 = 2 * i
                c1 = c0 + 1
                wait_fire(r0, gsem0)
                wb(c0, r0, wsem0)

                @pl.when(c1 < ncht)
                def _():
                    wait_fire(r1, gsem1)
                    wb(c1, r1, wsem1)

                wait_wb(r0, wsem0)

                @pl.when(c0 + 2 < ncht)
                def _():
                    fire(c0 + 2, r0, gsem0)

                @pl.when(c1 < ncht)
                def _():
                    wait_wb(r1, wsem1)

                    @pl.when(c1 + 2 < ncht)
                    def _():
                        fire(c1 + 2, r1, gsem1)

                return carry

            lax.fori_loop(0, (ncht + 1) // 2, body, 0)

        @pl.when(cid == 0)
        def _():
            run_stream(ps_hbm, send_hbm, gs_hbm)

        @pl.when(cid == 1)
        def _():
            run_stream(pr_hbm, recv_hbm, gr_hbm)

    @functools.partial(
        pl.kernel,
        out_type=[jax.ShapeDtypeStruct((NP, H), _f32),
                  jax.ShapeDtypeStruct((NP, H), _f32)],
        mesh=mesh,
        scratch_types=[
            pltpu.VMEM((epw,), jnp.int32),
            pltpu.VMEM((chunk, H), _f32),
            pltpu.VMEM((chunk, H), _f32),
            pltpu.VMEM_SHARED((NP, H), _f32),
        ] + [pltpu.SemaphoreType.DMA] * 4,
    )
    def sc_scatter(ue_hbm, recv_hbm, zeros_hbm, p0_hbm, p1_hbm,
                   idx_v, r0, r1, acc,
                   lsem0, lsem1, asem0, asem1):
        cid = lax.axis_index("c")
        sid = lax.axis_index("s")
        row0 = sid * ROWS_PER_TILE
        pltpu.sync_copy(zeros_hbm.at[pl.ds(row0, ROWS_PER_TILE)],
                        acc.at[pl.ds(row0, ROWS_PER_TILE)])

        base = cid * (ne // NC) + sid * epw
        pltpu.sync_copy(recv_hbm.at[pl.ds(base, epw)], idx_v)
        plsc.subcore_barrier()

        def load(c, r, sem):
            off = pl.multiple_of(base + c * chunk, 8)
            pltpu.async_copy(ue_hbm.at[pl.ds(off, chunk)], r, sem)

        def wait_load(r, sem):
            pltpu.make_async_copy(ue_hbm.at[pl.ds(0, chunk)], r, sem).wait()

        def add(c, r, sem):
            off = pl.multiple_of(c * chunk, 8)
            pltpu.async_copy(r, acc.at[idx_v.at[pl.ds(off, chunk)]], sem,
                             add=True)

        def wait_add(r, sem):
            pltpu.make_async_copy(r, acc.at[pl.ds(0, chunk)], sem).wait()

        load(0, r0, lsem0)
        load(1, r1, lsem1)

        def body(i, carry):
            c0 = 2 * i
            c1 = c0 + 1
            wait_load(r0, lsem0)
            add(c0, r0, asem0)

            @pl.when(c1 < nchunk)
            def _():
                wait_load(r1, lsem1)
                add(c1, r1, asem1)

            wait_add(r0, asem0)

            @pl.when(c0 + 2 < nchunk)
            def _():
                load(c0 + 2, r0, lsem0)

            @pl.when(c1 < nchunk)
            def _():
                wait_add(r1, asem1)

                @pl.when(c1 + 2 < nchunk)
                def _():
                    load(c1 + 2, r1, lsem1)

            return carry

        lax.fori_loop(0, (nchunk + 1) // 2, body, 0)
        plsc.subcore_barrier()

        @pl.when(cid == 0)
        def _():
            pltpu.sync_copy(acc.at[pl.ds(row0, ROWS_PER_TILE)],
                            p0_hbm.at[pl.ds(row0, ROWS_PER_TILE)])

        @pl.when(cid == 1)
        def _():
            pltpu.sync_copy(acc.at[pl.ds(row0, ROWS_PER_TILE)],
                            p1_hbm.at[pl.ds(row0, ROWS_PER_TILE)])

    return sc_gather, sc_scatter


def kernel(node_features, edge_index, edge_attr, edge_params, node_params):
    (w1, b1), (w2, b2), (w3, b3), (w4, b4), g, beta = edge_params
    (v1, c1), (v2, c2), (v3, c3), (v4, c4), gn, bn = node_params

    send = edge_index[0].astype(jnp.int32)
    recv = edge_index[1].astype(jnp.int32)

    w1s, w1r, w1e = w1[:H], w1[H:2 * H], w1[2 * H:]
    v1a, v1b = v1[:H], v1[H:]
    row = lambda v: v.reshape(1, H)

    bn_rows = 1000
    ps, pr = pl.pallas_call(
        _precompute_body,
        grid=(N // bn_rows,),
        in_specs=[_row_spec(bn_rows), _const_spec((H, H)), _const_spec((H, H)),
                  _const_spec((1, H))],
        out_specs=[_row_spec(bn_rows), _row_spec(bn_rows)],
        out_shape=[jax.ShapeDtypeStruct((NP, H), _f32),
                   jax.ShapeDtypeStruct((NP, H), _f32)],
    )(node_features, w1s, w1r, row(b1))

    zeros = jnp.zeros((NP, H), _f32)
    ew = (w1e.astype(_bf16), w2.astype(_bf16), row(b2), w3.astype(_bf16),
          row(b3), w4.astype(_bf16), row(b4), row(g), row(beta))

    be_rows = 2000

    def edge_mlp(gs, gr, ne, blk_off, oe_prev):
        nblk = ne // be_rows
        base_specs = [_row_spec(be_rows), _row_spec(be_rows),
                      _row_spec(be_rows, off=blk_off),
                      _const_spec((H, H)),
                      _const_spec((H, H)), _const_spec((1, H)),
                      _const_spec((H, H)), _const_spec((1, H)),
                      _const_spec((H, H)), _const_spec((1, H)),
                      _const_spec((1, H)), _const_spec((1, H))]
        out_specs = [_row_spec(be_rows), _row_spec(be_rows, off=blk_off)]
        out_shape = [jax.ShapeDtypeStruct((ne, H), _f32),
                     jax.ShapeDtypeStruct((E, H), _f32)]
        if oe_prev is None:
            return pl.pallas_call(
                _edge_mlp_body, grid=(nblk,), in_specs=base_specs,
                out_specs=out_specs, out_shape=out_shape,
            )(gs, gr, edge_attr, *ew)
        return pl.pallas_call(
            _edge_mlp_body2, grid=(nblk,),
            in_specs=base_specs + [pl.BlockSpec(memory_space=pl.ANY)],
            out_specs=out_specs, out_shape=out_shape,
            input_output_aliases={12: 1},
        )(gs, gr, edge_attr, *ew, oe_prev)

    offs = [0]
    for ne in SPLITS:
        offs.append(offs[-1] + ne)
    gathered = []
    for i, ne in enumerate(SPLITS):
        sc_gather, _ = _sc_kernels(ne)
        e0, e1 = offs[i], offs[i + 1]
        gathered.append(sc_gather(ps, pr, send[e0:e1], recv[e0:e1]))

    oe = None
    ues = []
    for i, ne in enumerate(SPLITS):
        gs, gr = gathered[i]
        ue, oe = edge_mlp(gs, gr, ne, offs[i] // be_rows, oe)
        ues.append(ue)
    out_edges = oe

    parts = []
    for i, ne in enumerate(SPLITS):
        _, sc_scatter = _sc_kernels(ne)
        e0, e1 = offs[i], offs[i + 1]
        parts.extend(sc_scatter(ues[i], recv[e0:e1], zeros))

    out_nodes = pl.pallas_call(
        _node_mlp_body,
        grid=(N // bn_rows,),
        in_specs=[_row_spec(bn_rows)] * (1 + _NPART) + [
                  _const_spec((H, H)), _const_spec((H, H)), _const_spec((1, H)),
                  _const_spec((H, H)), _const_spec((1, H)),
                  _const_spec((H, H)), _const_spec((1, H)),
                  _const_spec((H, H)), _const_spec((1, H)),
                  _const_spec((1, H)), _const_spec((1, H))],
        out_specs=_row_spec(bn_rows),
        out_shape=jax.ShapeDtypeStruct((N, H), _f32),
    )(node_features, *parts, v1a, v1b, row(c1), v2, row(c2),
      v3, row(c3), v4, row(c4), row(gn), row(bn))

    return (out_nodes, edge_index, out_edges)


# speedup vs baseline: 1.0825x; 1.0825x over previous
import functools

import jax
import jax.numpy as jnp
from jax import lax
from jax.experimental import pallas as pl
from jax.experimental.pallas import tpu as pltpu
from jax.experimental.pallas import tpu_sc as plsc

H = 128
N = 10000
E = 320000
SPLITS = (64000, 96000, 96000, 64000)

NC = 2
NS = 16
NW = NC * NS
NP = 10240
ROWS_PER_TILE = NP // NS

_f32 = jnp.float32
_bf16 = jnp.bfloat16


def _pick_chunk(n):
    for c in range(128, 0, -8):
        if n % c == 0:
            return c
    raise ValueError(n)


def _precompute_body(x, w1s, w1r, b1, ps, pr):
    xv = x[...]
    ps[...] = jnp.dot(xv, w1s[...], preferred_element_type=_f32) + b1[...]
    pr[...] = jnp.dot(xv, w1r[...], preferred_element_type=_f32)


def _bdot(a, b):
    return jnp.dot(a.astype(_bf16), b, preferred_element_type=_f32)


def _edge_mlp_body(gs, gr, ea, w1e, w2, b2, w3, b3, w4, b4, g, beta,
                   ue, oe):
    eav = ea[...]
    h = (gs[...] + gr[...] + _bdot(eav, w1e[...]))
    h = jnp.maximum(h, 0.0)
    h = jnp.maximum(_bdot(h, w2[...]) + b2[...], 0.0)
    h = jnp.maximum(_bdot(h, w3[...]) + b3[...], 0.0)
    h = _bdot(h, w4[...]) + b4[...]
    mu = jnp.mean(h, axis=1, keepdims=True)
    d = h - mu
    var = jnp.mean(d * d, axis=1, keepdims=True)
    u = d * lax.rsqrt(var + 1e-5) * g[...] + beta[...]
    ue[...] = u
    oe[...] = eav + u


def _edge_mlp_body2(gs, gr, ea, w1e, w2, b2, w3, b3, w4, b4, g, beta, _oe_in,
                    ue, oe):
    _edge_mlp_body(gs, gr, ea, w1e, w2, b2, w3, b3, w4, b4, g, beta, ue, oe)


_NPART = 2 * len(SPLITS)


def _node_mlp_body(x, *args):
    parts = args[:_NPART]
    v1a, v1b, c1, v2, c2, v3, c3, v4, c4, gn, bn, out = args[_NPART:]
    xv = x[...]
    agg = parts[0][...]
    for p in parts[1:]:
        agg = agg + p[...]
    h = (jnp.dot(xv, v1a[...], preferred_element_type=_f32)
         + jnp.dot(agg, v1b[...], preferred_element_type=_f32) + c1[...])
    h = jnp.maximum(h, 0.0)
    h = jnp.maximum(jnp.dot(h, v2[...], preferred_element_type=_f32) + c2[...], 0.0)
    h = jnp.maximum(jnp.dot(h, v3[...], preferred_element_type=_f32) + c3[...], 0.0)
    h = jnp.dot(h, v4[...], preferred_element_type=_f32) + c4[...]
    mu = jnp.mean(h, axis=1, keepdims=True)
    d = h - mu
    var = jnp.mean(d * d, axis=1, keepdims=True)
    out[...] = xv + d * lax.rsqrt(var + 1e-5) * gn[...] + bn[...]


def _row_spec(block_rows, off=0):
    return pl.BlockSpec((block_rows, H), lambda i: (i + off, 0))


def _const_spec(shape):
    return pl.BlockSpec(shape, lambda i: (0, 0))


@functools.cache
def _sc_kernels(ne):
    epw = ne // NW
    chunk = _pick_chunk(epw)
    nchunk = epw // chunk
    mesh = plsc.VectorSubcoreMesh(core_axis_name="c", subcore_axis_name="s",
                                  num_cores=NC, num_subcores=NS)

    ept = ne // NS
    cht = _pick_chunk(ept)
    ncht = ept // cht

    @functools.partial(
        pl.kernel,
        out_type=[jax.ShapeDtypeStruct((ne, H), _f32),
                  jax.ShapeDtypeStruct((ne, H), _f32)],
        mesh=mesh,
        scratch_types=[
            pltpu.VMEM((ept,), jnp.int32),
            pltpu.VMEM((cht, H), _f32),
            pltpu.VMEM((cht, H), _f32),
            pltpu.VMEM_SHARED((NP, H), _f32),
        ] + [pltpu.SemaphoreType.DMA] * 4,
    )
    def sc_gather(ps_hbm, pr_hbm, send_hbm, recv_hbm, gs_hbm, gr_hbm,
                  idx, r0, r1, tbl, gsem0, gsem1, wsem0, wsem1):
        cid = lax.axis_index("c")
        sid = lax.axis_index("s")
        base = sid * ept

        def run_stream(tbl_hbm, sidx_hbm, out_hbm):
            pltpu.sync_copy(tbl_hbm.at[pl.ds(sid * ROWS_PER_TILE,
                                             ROWS_PER_TILE)],
                            tbl.at[pl.ds(sid * ROWS_PER_TILE,
                                         ROWS_PER_TILE)])
            pltpu.sync_copy(sidx_hbm.at[pl.ds(base, ept)], idx)
            plsc.subcore_barrier()

            def fire(c, r, sem):
                off = pl.multiple_of(c * cht, 8)
                pltpu.async_copy(tbl.at[idx.at[pl.ds(off, cht)]], r, sem)

            def wait_fire(r, sem):
                pltpu.make_async_copy(tbl.at[pl.ds(0, cht)], r, sem).wait()

            def wb(c, r, sem):
                off = pl.multiple_of(base + c * cht, 8)
                pltpu.async_copy(r, out_hbm.at[pl.ds(off, cht)], sem)

            def wait_wb(r, sem):
                pltpu.make_async_copy(r, out_hbm.at[pl.ds(0, cht)],
                                      sem).wait()

            fire(0, r0, gsem0)
            fire(1, r1, gsem1)

            def body(i, carry):
                c0 = 2 * i
                c1 = c0 + 1
                wait_fire(r0, gsem0)
                wb(c0, r0, wsem0)

                @pl.when(c1 < ncht)
                def _():
                    wait_fire(r1, gsem1)
                    wb(c1, r1, wsem1)

                wait_wb(r0, wsem0)

                @pl.when(c0 + 2 < ncht)
                def _():
                    fire(c0 + 2, r0, gsem0)

                @pl.when(c1 < ncht)
                def _():
                    wait_wb(r1, wsem1)

                    @pl.when(c1 + 2 < ncht)
                    def _():
                        fire(c1 + 2, r1, gsem1)

                return carry

            lax.fori_loop(0, (ncht + 1) // 2, body, 0)

        @pl.when(cid == 0)
        def _():
            run_stream(ps_hbm, send_hbm, gs_hbm)

        @pl.when(cid == 1)
        def _():
            run_stream(pr_hbm, recv_hbm, gr_hbm)

    @functools.partial(
        pl.kernel,
        out_type=[jax.ShapeDtypeStruct((NP, H), _f32),
                  jax.ShapeDtypeStruct((NP, H), _f32)],
        mesh=mesh,
        scratch_types=[
            pltpu.VMEM((epw,), jnp.int32),
            pltpu.VMEM((chunk, H), _f32),
            pltpu.VMEM((chunk, H), _f32),
            pltpu.VMEM_SHARED((NP, H), _f32),
        ] + [pltpu.SemaphoreType.DMA] * 4,
    )
    def sc_scatter(ue_hbm, recv_hbm, p0_hbm, p1_hbm,
                   idx_v, r0, r1, acc,
                   lsem0, lsem1, asem0, asem1):
        cid = lax.axis_index("c")
        sid = lax.axis_index("s")
        row0 = sid * ROWS_PER_TILE

        def zr(k, carry):
            def zc(j, carry2):
                r0[k, pl.ds(j * 16, 16)] = jnp.zeros((16,), _f32)
                return carry2
            return lax.fori_loop(0, H // 16, zc, carry)

        lax.fori_loop(0, 80, zr, 0)

        def zcopy(i, carry):
            pltpu.sync_copy(r0.at[pl.ds(0, 80)],
                            acc.at[pl.ds(row0 + i * 80, 80)])
            return carry

        lax.fori_loop(0, ROWS_PER_TILE // 80, zcopy, 0)

        base = cid * (ne // NC) + sid * epw
        pltpu.sync_copy(recv_hbm.at[pl.ds(base, epw)], idx_v)
        plsc.subcore_barrier()

        def load(c, r, sem):
            off = pl.multiple_of(base + c * chunk, 8)
            pltpu.async_copy(ue_hbm.at[pl.ds(off, chunk)], r, sem)

        def wait_load(r, sem):
            pltpu.make_async_copy(ue_hbm.at[pl.ds(0, chunk)], r, sem).wait()

        def add(c, r, sem):
            off = pl.multiple_of(c * chunk, 8)
            pltpu.async_copy(r, acc.at[idx_v.at[pl.ds(off, chunk)]], sem,
                             add=True)

        def wait_add(r, sem):
            pltpu.make_async_copy(r, acc.at[pl.ds(0, chunk)], sem).wait()

        load(0, r0, lsem0)
        load(1, r1, lsem1)

        def body(i, carry):
            c0 = 2 * i
            c1 = c0 + 1
            wait_load(r0, lsem0)
            add(c0, r0, asem0)

            @pl.when(c1 < nchunk)
            def _():
                wait_load(r1, lsem1)
                add(c1, r1, asem1)

            wait_add(r0, asem0)

            @pl.when(c0 + 2 < nchunk)
            def _():
                load(c0 + 2, r0, lsem0)

            @pl.when(c1 < nchunk)
            def _():
                wait_add(r1, asem1)

                @pl.when(c1 + 2 < nchunk)
                def _():
                    load(c1 + 2, r1, lsem1)

            return carry

        lax.fori_loop(0, (nchunk + 1) // 2, body, 0)
        plsc.subcore_barrier()

        @pl.when(cid == 0)
        def _():
            pltpu.sync_copy(acc.at[pl.ds(row0, ROWS_PER_TILE)],
                            p0_hbm.at[pl.ds(row0, ROWS_PER_TILE)])

        @pl.when(cid == 1)
        def _():
            pltpu.sync_copy(acc.at[pl.ds(row0, ROWS_PER_TILE)],
                            p1_hbm.at[pl.ds(row0, ROWS_PER_TILE)])

    return sc_gather, sc_scatter


def kernel(node_features, edge_index, edge_attr, edge_params, node_params):
    (w1, b1), (w2, b2), (w3, b3), (w4, b4), g, beta = edge_params
    (v1, c1), (v2, c2), (v3, c3), (v4, c4), gn, bn = node_params

    send = edge_index[0].astype(jnp.int32)
    recv = edge_index[1].astype(jnp.int32)

    w1s, w1r, w1e = w1[:H], w1[H:2 * H], w1[2 * H:]
    v1a, v1b = v1[:H], v1[H:]
    row = lambda v: v.reshape(1, H)

    bn_rows = 1000
    ps, pr = pl.pallas_call(
        _precompute_body,
        grid=(N // bn_rows,),
        in_specs=[_row_spec(bn_rows), _const_spec((H, H)), _const_spec((H, H)),
                  _const_spec((1, H))],
        out_specs=[_row_spec(bn_rows), _row_spec(bn_rows)],
        out_shape=[jax.ShapeDtypeStruct((NP, H), _f32),
                   jax.ShapeDtypeStruct((NP, H), _f32)],
    )(node_features, w1s, w1r, row(b1))

    ew = (w1e.astype(_bf16), w2.astype(_bf16), row(b2), w3.astype(_bf16),
          row(b3), w4.astype(_bf16), row(b4), row(g), row(beta))

    be_rows = 2000

    def edge_mlp(gs, gr, ne, blk_off, oe_prev):
        nblk = ne // be_rows
        base_specs = [_row_spec(be_rows), _row_spec(be_rows),
                      _row_spec(be_rows, off=blk_off),
                      _const_spec((H, H)),
                      _const_spec((H, H)), _const_spec((1, H)),
                      _const_spec((H, H)), _const_spec((1, H)),
                      _const_spec((H, H)), _const_spec((1, H)),
                      _const_spec((1, H)), _const_spec((1, H))]
        out_specs = [_row_spec(be_rows), _row_spec(be_rows, off=blk_off)]
        out_shape = [jax.ShapeDtypeStruct((ne, H), _f32),
                     jax.ShapeDtypeStruct((E, H), _f32)]
        if oe_prev is None:
            return pl.pallas_call(
                _edge_mlp_body, grid=(nblk,), in_specs=base_specs,
                out_specs=out_specs, out_shape=out_shape,
            )(gs, gr, edge_attr, *ew)
        return pl.pallas_call(
            _edge_mlp_body2, grid=(nblk,),
            in_specs=base_specs + [pl.BlockSpec(memory_space=pl.ANY)],
            out_specs=out_specs, out_shape=out_shape,
            input_output_aliases={12: 1},
        )(gs, gr, edge_attr, *ew, oe_prev)

    offs = [0]
    for ne in SPLITS:
        offs.append(offs[-1] + ne)
    gathered = []
    for i, ne in enumerate(SPLITS):
        sc_gather, _ = _sc_kernels(ne)
        e0, e1 = offs[i], offs[i + 1]
        gathered.append(sc_gather(ps, pr, send[e0:e1], recv[e0:e1]))

    oe = None
    ues = []
    for i, ne in enumerate(SPLITS):
        gs, gr = gathered[i]
        ue, oe = edge_mlp(gs, gr, ne, offs[i] // be_rows, oe)
        ues.append(ue)
    out_edges = oe

    parts = []
    for i, ne in enumerate(SPLITS):
        _, sc_scatter = _sc_kernels(ne)
        e0, e1 = offs[i], offs[i + 1]
        parts.extend(sc_scatter(ues[i], recv[e0:e1]))

    out_nodes = pl.pallas_call(
        _node_mlp_body,
        grid=(N // bn_rows,),
        in_specs=[_row_spec(bn_rows)] * (1 + _NPART) + [
                  _const_spec((H, H)), _const_spec((H, H)), _const_spec((1, H)),
                  _const_spec((H, H)), _const_spec((1, H)),
                  _const_spec((H, H)), _const_spec((1, H)),
                  _const_spec((H, H)), _const_spec((1, H)),
                  _const_spec((1, H)), _const_spec((1, H))],
        out_specs=_row_spec(bn_rows),
        out_shape=jax.ShapeDtypeStruct((N, H), _f32),
    )(node_features, *parts, v1a, v1b, row(c1), v2, row(c2),
      v3, row(c3), v4, row(c4), row(gn), row(bn))

    return (out_nodes, edge_index, out_edges)


# speedup vs baseline: 1.0953x; 1.0118x over previous
import functools

import jax
import jax.numpy as jnp
from jax import lax
from jax.experimental import pallas as pl
from jax.experimental.pallas import tpu as pltpu
from jax.experimental.pallas import tpu_sc as plsc

H = 128
N = 10000
E = 320000
SPLITS = (64000, 96000, 96000, 64000)

NC = 2
NS = 16
NW = NC * NS
NP = 10240
ROWS_PER_TILE = NP // NS

_f32 = jnp.float32
_bf16 = jnp.bfloat16


def _pick_chunk(n):
    for c in range(128, 0, -8):
        if n % c == 0:
            return c
    raise ValueError(n)


def _precompute_body(x, w1s, w1r, b1, ps, pr):
    xv = x[...]
    ps[...] = jnp.dot(xv, w1s[...], preferred_element_type=_f32) + b1[...]
    pr[...] = jnp.dot(xv, w1r[...], preferred_element_type=_f32)


def _bdot(a, b):
    return jnp.dot(a.astype(_bf16), b, preferred_element_type=_f32)


def _edge_mlp_body(gs, gr, ea, w1e, w2, b2, w3, b3, w4, b4, g, beta,
                   ue, oe):
    eav = ea[...]
    h = (gs[...] + gr[...] + _bdot(eav, w1e[...]))
    h = jnp.maximum(h, 0.0)
    h = jnp.maximum(_bdot(h, w2[...]) + b2[...], 0.0)
    h = jnp.maximum(_bdot(h, w3[...]) + b3[...], 0.0)
    h = _bdot(h, w4[...]) + b4[...]
    mu = jnp.mean(h, axis=1, keepdims=True)
    d = h - mu
    var = jnp.mean(d * d, axis=1, keepdims=True)
    u = d * lax.rsqrt(var + 1e-5) * g[...] + beta[...]
    ue[...] = u
    oe[...] = eav + u


def _edge_mlp_body2(gs, gr, ea, w1e, w2, b2, w3, b3, w4, b4, g, beta, _oe_in,
                    ue, oe):
    _edge_mlp_body(gs, gr, ea, w1e, w2, b2, w3, b3, w4, b4, g, beta, ue, oe)


_NPART = 2 * len(SPLITS)


def _node_mlp_body(x, *args):
    parts = args[:_NPART]
    v1a, v1b, c1, v2, c2, v3, c3, v4, c4, gn, bn, out = args[_NPART:]
    xv = x[...]
    agg = parts[0][...]
    for p in parts[1:]:
        agg = agg + p[...]
    h = (jnp.dot(xv, v1a[...], preferred_element_type=_f32)
         + jnp.dot(agg, v1b[...], preferred_element_type=_f32) + c1[...])
    h = jnp.maximum(h, 0.0)
    h = jnp.maximum(jnp.dot(h, v2[...], preferred_element_type=_f32) + c2[...], 0.0)
    h = jnp.maximum(jnp.dot(h, v3[...], preferred_element_type=_f32) + c3[...], 0.0)
    h = jnp.dot(h, v4[...], preferred_element_type=_f32) + c4[...]
    mu = jnp.mean(h, axis=1, keepdims=True)
    d = h - mu
    var = jnp.mean(d * d, axis=1, keepdims=True)
    out[...] = xv + d * lax.rsqrt(var + 1e-5) * gn[...] + bn[...]


def _row_spec(block_rows, off=0):
    return pl.BlockSpec((block_rows, H), lambda i: (i + off, 0))


def _const_spec(shape):
    return pl.BlockSpec(shape, lambda i: (0, 0))


@functools.cache
def _sc_kernels(ne):
    epw = ne // NW
    chunk = _pick_chunk(epw)
    nchunk = epw // chunk
    mesh = plsc.VectorSubcoreMesh(core_axis_name="c", subcore_axis_name="s",
                                  num_cores=NC, num_subcores=NS)

    ept = ne // NS
    cht = _pick_chunk(ept)
    ncht = ept // cht

    @functools.partial(
        pl.kernel,
        out_type=[jax.ShapeDtypeStruct((ne, H), _f32),
                  jax.ShapeDtypeStruct((ne, H), _f32)],
        mesh=mesh,
        scratch_types=[
            pltpu.VMEM((ept,), jnp.int32),
            pltpu.VMEM((cht, H), _f32),
            pltpu.VMEM((cht, H), _f32),
            pltpu.VMEM_SHARED((NP, H), _f32),
        ] + [pltpu.SemaphoreType.DMA] * 4,
    )
    def sc_gather(ps_hbm, pr_hbm, send_hbm, recv_hbm, gs_hbm, gr_hbm,
                  idx, r0, r1, tbl, gsem0, gsem1, wsem0, wsem1):
        cid = lax.axis_index("c")
        sid = lax.axis_index("s")
        base = sid * ept

        def run_stream(tbl_hbm, sidx_hbm, out_hbm):
            pltpu.sync_copy(tbl_hbm.at[pl.ds(sid * ROWS_PER_TILE,
                                             ROWS_PER_TILE)],
                            tbl.at[pl.ds(sid * ROWS_PER_TILE,
                                         ROWS_PER_TILE)])
            pltpu.sync_copy(sidx_hbm.at[pl.ds(base, ept)], idx)
            plsc.subcore_barrier()

            def fire(c, r, sem):
                off = pl.multiple_of(c * cht, 8)
                pltpu.async_copy(tbl.at[idx.at[pl.ds(off, cht)]], r, sem)

            def wait_fire(r, sem):
                pltpu.make_async_copy(tbl.at[pl.ds(0, cht)], r, sem).wait()

            def wb(c, r, sem):
                off = pl.multiple_of(base + c * cht, 8)
                pltpu.async_copy(r, out_hbm.at[pl.ds(off, cht)], sem)

            def wait_wb(r, sem):
                pltpu.make_async_copy(r, out_hbm.at[pl.ds(0, cht)],
                                      sem).wait()

            fire(0, r0, gsem0)
            fire(1, r1, gsem1)

            def body(i, carry):
                c0 = 2 * i
                c1 = c0 + 1
                wait_fire(r0, gsem0)
                wb(c0, r0, wsem0)

                @pl.when(c1 < ncht)
                def _():
                    wait_fire(r1, gsem1)
                    wb(c1, r1, wsem1)

                wait_wb(r0, wsem0)

                @pl.when(c0 + 2 < ncht)
                def _():
                    fire(c0 + 2, r0, gsem0)

                @pl.when(c1 < ncht)
                def _():
                    wait_wb(r1, wsem1)

                    @pl.when(c1 + 2 < ncht)
                    def _():
                        fire(c1 + 2, r1, gsem1)

                return carry

            lax.fori_loop(0, (ncht + 1) // 2, body, 0)

        @pl.when(cid == 0)
        def _():
            run_stream(ps_hbm, send_hbm, gs_hbm)

        @pl.when(cid == 1)
        def _():
            run_stream(pr_hbm, recv_hbm, gr_hbm)

    NSLOT = 4
    schunk = 40
    assert epw % schunk == 0
    snchunk = epw // schunk

    @functools.partial(
        pl.kernel,
        out_type=[jax.ShapeDtypeStruct((NP, H), _f32),
                  jax.ShapeDtypeStruct((NP, H), _f32)],
        mesh=mesh,
        scratch_types=[
            pltpu.VMEM((epw,), jnp.int32),
        ] + [pltpu.VMEM((schunk, H), _f32)] * NSLOT + [
            pltpu.VMEM_SHARED((NP, H), _f32),
        ] + [pltpu.SemaphoreType.DMA] * (2 * NSLOT),
    )
    def sc_scatter(ue_hbm, recv_hbm, p0_hbm, p1_hbm, idx_v, *scr):
        rb = scr[:NSLOT]
        acc = scr[NSLOT]
        lsems = scr[NSLOT + 1:2 * NSLOT + 1]
        asems = scr[2 * NSLOT + 1:]
        cid = lax.axis_index("c")
        sid = lax.axis_index("s")
        row0 = sid * ROWS_PER_TILE

        def zr(k, carry):
            def zc(j, carry2):
                rb[0][k, pl.ds(j * 16, 16)] = jnp.zeros((16,), _f32)
                return carry2
            return lax.fori_loop(0, H // 16, zc, carry)

        lax.fori_loop(0, 80, zr, 0)

        def zcopy(i, carry):
            pltpu.sync_copy(rb[0].at[pl.ds(0, 80)],
                            acc.at[pl.ds(row0 + i * 80, 80)])
            return carry

        lax.fori_loop(0, ROWS_PER_TILE // 80, zcopy, 0)

        base = cid * (ne // NC) + sid * epw
        pltpu.sync_copy(recv_hbm.at[pl.ds(base, epw)], idx_v)
        plsc.subcore_barrier()

        def load(c, j):
            off = pl.multiple_of(base + c * schunk, 8)
            pltpu.async_copy(ue_hbm.at[pl.ds(off, schunk)], rb[j], lsems[j])

        def wait_load(j):
            pltpu.make_async_copy(ue_hbm.at[pl.ds(0, schunk)], rb[j],
                                  lsems[j]).wait()

        def add(c, j):
            off = pl.multiple_of(c * schunk, 8)
            pltpu.async_copy(rb[j], acc.at[idx_v.at[pl.ds(off, schunk)]],
                             asems[j], add=True)

        def wait_add(j):
            pltpu.make_async_copy(rb[j], acc.at[pl.ds(0, schunk)],
                                  asems[j]).wait()

        for j in range(NSLOT):
            load(j, j)

        def body(i, carry):
            c0 = NSLOT * i

            def stage1(j):
                @pl.when(c0 + j < snchunk)
                def _():
                    wait_load(j)
                    add(c0 + j, j)

            def stage2(j):
                @pl.when(c0 + j < snchunk)
                def _():
                    wait_add(j)

                    @pl.when(c0 + j + NSLOT < snchunk)
                    def _():
                        load(c0 + j + NSLOT, j)

            for j in range(NSLOT):
                stage1(j)
            for j in range(NSLOT):
                stage2(j)
            return carry

        lax.fori_loop(0, (snchunk + NSLOT - 1) // NSLOT, body, 0)
        plsc.subcore_barrier()

        @pl.when(cid == 0)
        def _():
            pltpu.sync_copy(acc.at[pl.ds(row0, ROWS_PER_TILE)],
                            p0_hbm.at[pl.ds(row0, ROWS_PER_TILE)])

        @pl.when(cid == 1)
        def _():
            pltpu.sync_copy(acc.at[pl.ds(row0, ROWS_PER_TILE)],
                            p1_hbm.at[pl.ds(row0, ROWS_PER_TILE)])

    return sc_gather, sc_scatter


def kernel(node_features, edge_index, edge_attr, edge_params, node_params):
    (w1, b1), (w2, b2), (w3, b3), (w4, b4), g, beta = edge_params
    (v1, c1), (v2, c2), (v3, c3), (v4, c4), gn, bn = node_params

    send = edge_index[0].astype(jnp.int32)
    recv = edge_index[1].astype(jnp.int32)

    w1s, w1r, w1e = w1[:H], w1[H:2 * H], w1[2 * H:]
    v1a, v1b = v1[:H], v1[H:]
    row = lambda v: v.reshape(1, H)

    bn_rows = 1000
    ps, pr = pl.pallas_call(
        _precompute_body,
        grid=(N // bn_rows,),
        in_specs=[_row_spec(bn_rows), _const_spec((H, H)), _const_spec((H, H)),
                  _const_spec((1, H))],
        out_specs=[_row_spec(bn_rows), _row_spec(bn_rows)],
        out_shape=[jax.ShapeDtypeStruct((NP, H), _f32),
                   jax.ShapeDtypeStruct((NP, H), _f32)],
    )(node_features, w1s, w1r, row(b1))

    ew = (w1e.astype(_bf16), w2.astype(_bf16), row(b2), w3.astype(_bf16),
          row(b3), w4.astype(_bf16), row(b4), row(g), row(beta))

    be_rows = 2000

    def edge_mlp(gs, gr, ne, blk_off, oe_prev):
        nblk = ne // be_rows
        base_specs = [_row_spec(be_rows), _row_spec(be_rows),
                      _row_spec(be_rows, off=blk_off),
                      _const_spec((H, H)),
                      _const_spec((H, H)), _const_spec((1, H)),
                      _const_spec((H, H)), _const_spec((1, H)),
                      _const_spec((H, H)), _const_spec((1, H)),
                      _const_spec((1, H)), _const_spec((1, H))]
        out_specs = [_row_spec(be_rows), _row_spec(be_rows, off=blk_off)]
        out_shape = [jax.ShapeDtypeStruct((ne, H), _f32),
                     jax.ShapeDtypeStruct((E, H), _f32)]
        if oe_prev is None:
            return pl.pallas_call(
                _edge_mlp_body, grid=(nblk,), in_specs=base_specs,
                out_specs=out_specs, out_shape=out_shape,
            )(gs, gr, edge_attr, *ew)
        return pl.pallas_call(
            _edge_mlp_body2, grid=(nblk,),
            in_specs=base_specs + [pl.BlockSpec(memory_space=pl.ANY)],
            out_specs=out_specs, out_shape=out_shape,
            input_output_aliases={12: 1},
        )(gs, gr, edge_attr, *ew, oe_prev)

    offs = [0]
    for ne in SPLITS:
        offs.append(offs[-1] + ne)
    gathered = []
    for i, ne in enumerate(SPLITS):
        sc_gather, _ = _sc_kernels(ne)
        e0, e1 = offs[i], offs[i + 1]
        gathered.append(sc_gather(ps, pr, send[e0:e1], recv[e0:e1]))

    oe = None
    ues = []
    for i, ne in enumerate(SPLITS):
        gs, gr = gathered[i]
        ue, oe = edge_mlp(gs, gr, ne, offs[i] // be_rows, oe)
        ues.append(ue)
    out_edges = oe

    parts = []
    for i, ne in enumerate(SPLITS):
        _, sc_scatter = _sc_kernels(ne)
        e0, e1 = offs[i], offs[i + 1]
        parts.extend(sc_scatter(ues[i], recv[e0:e1]))

    out_nodes = pl.pallas_call(
        _node_mlp_body,
        grid=(N // bn_rows,),
        in_specs=[_row_spec(bn_rows)] * (1 + _NPART) + [
                  _const_spec((H, H)), _const_spec((H, H)), _const_spec((1, H)),
                  _const_spec((H, H)), _const_spec((1, H)),
                  _const_spec((H, H)), _const_spec((1, H)),
                  _const_spec((H, H)), _const_spec((1, H)),
                  _const_spec((1, H)), _const_spec((1, H))],
        out_specs=_row_spec(bn_rows),
        out_shape=jax.ShapeDtypeStruct((N, H), _f32),
    )(node_features, *parts, v1a, v1b, row(c1), v2, row(c2),
      v3, row(c3), v4, row(c4), row(gn), row(bn))

    return (out_nodes, edge_index, out_edges)


# speedup vs baseline: 1.1245x; 1.0267x over previous
import functools

import jax
import jax.numpy as jnp
from jax import lax
from jax.experimental import pallas as pl
from jax.experimental.pallas import tpu as pltpu
from jax.experimental.pallas import tpu_sc as plsc

H = 128
N = 10000
E = 320000
SPLITS = (64000, 96000, 96000, 64000)

NC = 2
NS = 16
NW = NC * NS
NP = 10240
ROWS_PER_TILE = NP // NS

_f32 = jnp.float32
_bf16 = jnp.bfloat16


def _pick_chunk(n):
    for c in range(128, 0, -8):
        if n % c == 0:
            return c
    raise ValueError(n)


def _precompute_body(x, w1s, w1r, b1, ps, pr):
    xv = x[...]
    ps[...] = jnp.dot(xv, w1s[...], preferred_element_type=_f32) + b1[...]
    pr[...] = jnp.dot(xv, w1r[...], preferred_element_type=_f32)


def _bdot(a, b):
    return jnp.dot(a.astype(_bf16), b, preferred_element_type=_f32)


def _edge_mlp_body(gs, gr, ea, w1e, w2, b2, w3, b3, w4, b4, g, beta,
                   ue, oe):
    eav = ea[...]
    h = (gs[...] + gr[...] + _bdot(eav, w1e[...]))
    h = jnp.maximum(h, 0.0)
    h = jnp.maximum(_bdot(h, w2[...]) + b2[...], 0.0)
    h = jnp.maximum(_bdot(h, w3[...]) + b3[...], 0.0)
    h = _bdot(h, w4[...]) + b4[...]
    mu = jnp.mean(h, axis=1, keepdims=True)
    d = h - mu
    var = jnp.mean(d * d, axis=1, keepdims=True)
    u = d * lax.rsqrt(var + 1e-5) * g[...] + beta[...]
    ue[...] = u
    oe[...] = eav + u


def _edge_mlp_body2(gs, gr, ea, w1e, w2, b2, w3, b3, w4, b4, g, beta, _oe_in,
                    ue, oe):
    _edge_mlp_body(gs, gr, ea, w1e, w2, b2, w3, b3, w4, b4, g, beta, ue, oe)


_NPART = 2 * len(SPLITS)


def _node_mlp_body(x, *args):
    parts = args[:_NPART]
    v1a, v1b, c1, v2, c2, v3, c3, v4, c4, gn, bn, out = args[_NPART:]
    xv = x[...]
    agg = parts[0][...]
    for p in parts[1:]:
        agg = agg + p[...]
    h = (jnp.dot(xv, v1a[...], preferred_element_type=_f32)
         + jnp.dot(agg, v1b[...], preferred_element_type=_f32) + c1[...])
    h = jnp.maximum(h, 0.0)
    h = jnp.maximum(jnp.dot(h, v2[...], preferred_element_type=_f32) + c2[...], 0.0)
    h = jnp.maximum(jnp.dot(h, v3[...], preferred_element_type=_f32) + c3[...], 0.0)
    h = jnp.dot(h, v4[...], preferred_element_type=_f32) + c4[...]
    mu = jnp.mean(h, axis=1, keepdims=True)
    d = h - mu
    var = jnp.mean(d * d, axis=1, keepdims=True)
    out[...] = xv + d * lax.rsqrt(var + 1e-5) * gn[...] + bn[...]


def _row_spec(block_rows, off=0):
    return pl.BlockSpec((block_rows, H), lambda i: (i + off, 0))


def _const_spec(shape):
    return pl.BlockSpec(shape, lambda i: (0, 0))


@functools.cache
def _sc_kernels(ne):
    epw = ne // NW
    chunk = _pick_chunk(epw)
    nchunk = epw // chunk
    mesh = plsc.VectorSubcoreMesh(core_axis_name="c", subcore_axis_name="s",
                                  num_cores=NC, num_subcores=NS)

    ept = ne // NS
    cht = 80
    assert ept % cht == 0
    ncht = ept // cht
    GSLOT = 4

    @functools.partial(
        pl.kernel,
        out_type=[jax.ShapeDtypeStruct((ne, H), _f32),
                  jax.ShapeDtypeStruct((ne, H), _f32)],
        mesh=mesh,
        scratch_types=[
            pltpu.VMEM((ept,), jnp.int32),
        ] + [pltpu.VMEM((cht, H), _f32)] * GSLOT + [
            pltpu.VMEM_SHARED((NP, H), _f32),
        ] + [pltpu.SemaphoreType.DMA] * (2 * GSLOT),
    )
    def sc_gather(ps_hbm, pr_hbm, send_hbm, recv_hbm, gs_hbm, gr_hbm,
                  idx, *scr):
        rb = scr[:GSLOT]
        tbl = scr[GSLOT]
        gsems = scr[GSLOT + 1:2 * GSLOT + 1]
        wsems = scr[2 * GSLOT + 1:]
        cid = lax.axis_index("c")
        sid = lax.axis_index("s")
        base = sid * ept

        def run_stream(tbl_hbm, sidx_hbm, out_hbm):
            pltpu.sync_copy(tbl_hbm.at[pl.ds(sid * ROWS_PER_TILE,
                                             ROWS_PER_TILE)],
                            tbl.at[pl.ds(sid * ROWS_PER_TILE,
                                         ROWS_PER_TILE)])
            pltpu.sync_copy(sidx_hbm.at[pl.ds(base, ept)], idx)
            plsc.subcore_barrier()

            def fire(c, j):
                off = pl.multiple_of(c * cht, 8)
                pltpu.async_copy(tbl.at[idx.at[pl.ds(off, cht)]], rb[j],
                                 gsems[j])

            def wait_fire(j):
                pltpu.make_async_copy(tbl.at[pl.ds(0, cht)], rb[j],
                                      gsems[j]).wait()

            def wb(c, j):
                off = pl.multiple_of(base + c * cht, 8)
                pltpu.async_copy(rb[j], out_hbm.at[pl.ds(off, cht)],
                                 wsems[j])

            def wait_wb(j):
                pltpu.make_async_copy(rb[j], out_hbm.at[pl.ds(0, cht)],
                                      wsems[j]).wait()

            for j in range(GSLOT):
                fire(j, j)

            def body(i, carry):
                c0 = GSLOT * i

                def stage1(j):
                    @pl.when(c0 + j < ncht)
                    def _():
                        wait_fire(j)
                        wb(c0 + j, j)

                def stage2(j):
                    @pl.when(c0 + j < ncht)
                    def _():
                        wait_wb(j)

                        @pl.when(c0 + j + GSLOT < ncht)
                        def _():
                            fire(c0 + j + GSLOT, j)

                for j in range(GSLOT):
                    stage1(j)
                for j in range(GSLOT):
                    stage2(j)
                return carry

            lax.fori_loop(0, (ncht + GSLOT - 1) // GSLOT, body, 0)

        @pl.when(cid == 0)
        def _():
            run_stream(ps_hbm, send_hbm, gs_hbm)

        @pl.when(cid == 1)
        def _():
            run_stream(pr_hbm, recv_hbm, gr_hbm)

    NSLOT = 4
    schunk = 40
    assert epw % schunk == 0
    snchunk = epw // schunk

    @functools.partial(
        pl.kernel,
        out_type=[jax.ShapeDtypeStruct((NP, H), _f32),
                  jax.ShapeDtypeStruct((NP, H), _f32)],
        mesh=mesh,
        scratch_types=[
            pltpu.VMEM((epw,), jnp.int32),
        ] + [pltpu.VMEM((schunk, H), _f32)] * NSLOT + [
            pltpu.VMEM_SHARED((NP, H), _f32),
        ] + [pltpu.SemaphoreType.DMA] * (2 * NSLOT),
    )
    def sc_scatter(ue_hbm, recv_hbm, p0_hbm, p1_hbm, idx_v, *scr):
        rb = scr[:NSLOT]
        acc = scr[NSLOT]
        lsems = scr[NSLOT + 1:2 * NSLOT + 1]
        asems = scr[2 * NSLOT + 1:]
        cid = lax.axis_index("c")
        sid = lax.axis_index("s")
        row0 = sid * ROWS_PER_TILE

        def zr(k, carry):
            def zc(j, carry2):
                rb[0][k, pl.ds(j * 16, 16)] = jnp.zeros((16,), _f32)
                return carry2
            return lax.fori_loop(0, H // 16, zc, carry)

        lax.fori_loop(0, 80, zr, 0)

        def zcopy(i, carry):
            pltpu.sync_copy(rb[0].at[pl.ds(0, 80)],
                            acc.at[pl.ds(row0 + i * 80, 80)])
            return carry

        lax.fori_loop(0, ROWS_PER_TILE // 80, zcopy, 0)

        base = cid * (ne // NC) + sid * epw
        pltpu.sync_copy(recv_hbm.at[pl.ds(base, epw)], idx_v)
        plsc.subcore_barrier()

        def load(c, j):
            off = pl.multiple_of(base + c * schunk, 8)
            pltpu.async_copy(ue_hbm.at[pl.ds(off, schunk)], rb[j], lsems[j])

        def wait_load(j):
            pltpu.make_async_copy(ue_hbm.at[pl.ds(0, schunk)], rb[j],
                                  lsems[j]).wait()

        def add(c, j):
            off = pl.multiple_of(c * schunk, 8)
            pltpu.async_copy(rb[j], acc.at[idx_v.at[pl.ds(off, schunk)]],
                             asems[j], add=True)

        def wait_add(j):
            pltpu.make_async_copy(rb[j], acc.at[pl.ds(0, schunk)],
                                  asems[j]).wait()

        for j in range(NSLOT):
            load(j, j)

        def body(i, carry):
            c0 = NSLOT * i

            def stage1(j):
                @pl.when(c0 + j < snchunk)
                def _():
                    wait_load(j)
                    add(c0 + j, j)

            def stage2(j):
                @pl.when(c0 + j < snchunk)
                def _():
                    wait_add(j)

                    @pl.when(c0 + j + NSLOT < snchunk)
                    def _():
                        load(c0 + j + NSLOT, j)

            for j in range(NSLOT):
                stage1(j)
            for j in range(NSLOT):
                stage2(j)
            return carry

        lax.fori_loop(0, (snchunk + NSLOT - 1) // NSLOT, body, 0)
        plsc.subcore_barrier()

        @pl.when(cid == 0)
        def _():
            pltpu.sync_copy(acc.at[pl.ds(row0, ROWS_PER_TILE)],
                            p0_hbm.at[pl.ds(row0, ROWS_PER_TILE)])

        @pl.when(cid == 1)
        def _():
            pltpu.sync_copy(acc.at[pl.ds(row0, ROWS_PER_TILE)],
                            p1_hbm.at[pl.ds(row0, ROWS_PER_TILE)])

    return sc_gather, sc_scatter


def kernel(node_features, edge_index, edge_attr, edge_params, node_params):
    (w1, b1), (w2, b2), (w3, b3), (w4, b4), g, beta = edge_params
    (v1, c1), (v2, c2), (v3, c3), (v4, c4), gn, bn = node_params

    send = edge_index[0].astype(jnp.int32)
    recv = edge_index[1].astype(jnp.int32)

    w1s, w1r, w1e = w1[:H], w1[H:2 * H], w1[2 * H:]
    v1a, v1b = v1[:H], v1[H:]
    row = lambda v: v.reshape(1, H)

    bn_rows = 1000
    ps, pr = pl.pallas_call(
        _precompute_body,
        grid=(N // bn_rows,),
        in_specs=[_row_spec(bn_rows), _const_spec((H, H)), _const_spec((H, H)),
                  _const_spec((1, H))],
        out_specs=[_row_spec(bn_rows), _row_spec(bn_rows)],
        out_shape=[jax.ShapeDtypeStruct((NP, H), _f32),
                   jax.ShapeDtypeStruct((NP, H), _f32)],
    )(node_features, w1s, w1r, row(b1))

    ew = (w1e.astype(_bf16), w2.astype(_bf16), row(b2), w3.astype(_bf16),
          row(b3), w4.astype(_bf16), row(b4), row(g), row(beta))

    be_rows = 2000

    def edge_mlp(gs, gr, ne, blk_off, oe_prev):
        nblk = ne // be_rows
        base_specs = [_row_spec(be_rows), _row_spec(be_rows),
                      _row_spec(be_rows, off=blk_off),
                      _const_spec((H, H)),
                      _const_spec((H, H)), _const_spec((1, H)),
                      _const_spec((H, H)), _const_spec((1, H)),
                      _const_spec((H, H)), _const_spec((1, H)),
                      _const_spec((1, H)), _const_spec((1, H))]
        out_specs = [_row_spec(be_rows), _row_spec(be_rows, off=blk_off)]
        out_shape = [jax.ShapeDtypeStruct((ne, H), _f32),
                     jax.ShapeDtypeStruct((E, H), _f32)]
        if oe_prev is None:
            return pl.pallas_call(
                _edge_mlp_body, grid=(nblk,), in_specs=base_specs,
                out_specs=out_specs, out_shape=out_shape,
            )(gs, gr, edge_attr, *ew)
        return pl.pallas_call(
            _edge_mlp_body2, grid=(nblk,),
            in_specs=base_specs + [pl.BlockSpec(memory_space=pl.ANY)],
            out_specs=out_specs, out_shape=out_shape,
            input_output_aliases={12: 1},
        )(gs, gr, edge_attr, *ew, oe_prev)

    offs = [0]
    for ne in SPLITS:
        offs.append(offs[-1] + ne)
    gathered = []
    for i, ne in enumerate(SPLITS):
        sc_gather, _ = _sc_kernels(ne)
        e0, e1 = offs[i], offs[i + 1]
        gathered.append(sc_gather(ps, pr, send[e0:e1], recv[e0:e1]))

    oe = None
    ues = []
    for i, ne in enumerate(SPLITS):
        gs, gr = gathered[i]
        ue, oe = edge_mlp(gs, gr, ne, offs[i] // be_rows, oe)
        ues.append(ue)
    out_edges = oe

    parts = []
    for i, ne in enumerate(SPLITS):
        _, sc_scatter = _sc_kernels(ne)
        e0, e1 = offs[i], offs[i + 1]
        parts.extend(sc_scatter(ues[i], recv[e0:e1]))

    out_nodes = pl.pallas_call(
        _node_mlp_body,
        grid=(N // bn_rows,),
        in_specs=[_row_spec(bn_rows)] * (1 + _NPART) + [
                  _const_spec((H, H)), _const_spec((H, H)), _const_spec((1, H)),
                  _const_spec((H, H)), _const_spec((1, H)),
                  _const_spec((H, H)), _const_spec((1, H)),
                  _const_spec((H, H)), _const_spec((1, H)),
                  _const_spec((1, H)), _const_spec((1, H))],
        out_specs=_row_spec(bn_rows),
        out_shape=jax.ShapeDtypeStruct((N, H), _f32),
    )(node_features, *parts, v1a, v1b, row(c1), v2, row(c2),
      v3, row(c3), v4, row(c4), row(gn), row(bn))

    return (out_nodes, edge_index, out_edges)
